# baseline (speedup 1.0000x reference)
import functools

import jax
import jax.numpy as jnp
from jax import lax
from jax.experimental import pallas as pl
from jax.experimental.pallas import tpu as pltpu
from jax.experimental.pallas import tpu_sc as plsc

_NC = 2
_NS = 16
_L = 16


def _front_body(x_ref, w_ref, asv_ref, adv_ref, ea_ref, w1b_ref, w2b_ref,
                ei_ref, h_ref, s_ref, d_ref, e1_ref, e2_ref, ei3_ref, *,
                one_col):
    ei3_ref[...] = ei_ref[...].reshape(ei3_ref.shape)
    h = jnp.dot(x_ref[...], w_ref[...], preferred_element_type=jnp.float32)
    col = lax.broadcasted_iota(jnp.int32, h.shape, 1)
    hp = h + jnp.where(col == one_col, 1.0, 0.0)
    h_ref[...] = hp
    s_ref[...] = jnp.sum(hp * asv_ref[...], axis=1, keepdims=True)
    d_ref[...] = jnp.sum(hp * adv_ref[...], axis=1, keepdims=True)
    ea = ea_ref[...]
    e1_ref[...] = jnp.dot(ea, w1b_ref[...], preferred_element_type=jnp.float32)
    e2_ref[...] = jnp.dot(ea, w2b_ref[...], preferred_element_type=jnp.float32)


def _tc_front(x, w1p, as1p, ad1p, ea3, wblk1, wblk2, ei, *, wp, one_col,
              ch, br=1000, bre=400):
    n, df = x.shape
    r, c = ea3.shape
    e = ei.shape[1]
    k = wblk1.shape[1]
    grid = (n // br,)
    be = e // (n // br)
    assert r // bre == n // br and be % ch == 0
    return pl.pallas_call(
        functools.partial(_front_body, one_col=one_col),
        grid=grid,
        in_specs=[
            pl.BlockSpec((br, df), lambda i: (i, 0)),
            pl.BlockSpec((df, wp), lambda i: (0, 0)),
            pl.BlockSpec((1, wp), lambda i: (0, 0)),
            pl.BlockSpec((1, wp), lambda i: (0, 0)),
            pl.BlockSpec((bre, c), lambda i: (i, 0)),
            pl.BlockSpec((c, k), lambda i: (0, 0)),
            pl.BlockSpec((c, k), lambda i: (0, 0)),
            pl.BlockSpec((2, be), lambda i: (0, i)),
        ],
        out_specs=[
            pl.BlockSpec((br, wp), lambda i: (i, 0)),
            pl.BlockSpec((br, 1), lambda i: (i, 0)),
            pl.BlockSpec((br, 1), lambda i: (i, 0)),
            pl.BlockSpec((bre, k), lambda i: (i, 0)),
            pl.BlockSpec((bre, k), lambda i: (i, 0)),
            pl.BlockSpec((2, be // ch, ch), lambda i: (0, i, 0)),
        ],
        out_shape=[
            jax.ShapeDtypeStruct((n, wp), jnp.float32),
            jax.ShapeDtypeStruct((n, 1), jnp.float32),
            jax.ShapeDtypeStruct((n, 1), jnp.float32),
            jax.ShapeDtypeStruct((r, k), jnp.float32),
            jax.ShapeDtypeStruct((r, k), jnp.float32),
            jax.ShapeDtypeStruct((2, e // ch, ch), jnp.int32),
        ],
    )(x, w1p, as1p, ad1p, ea3, wblk1, wblk2, ei)


def _mid_body(a_ref, b_ref, bias_ref, w2_ref, asv_ref, adv_ref,
              g_ref, s_ref, d_ref, *, den_col, one_col):
    s = a_ref[...] + b_ref[...]
    den = s[:, den_col:den_col + 1]
    o = s / (den + 1e-16)
    h2 = jnp.maximum(o + bias_ref[...], 0.0)
    g = jnp.dot(h2, w2_ref[...], preferred_element_type=jnp.float32)
    col = lax.broadcasted_iota(jnp.int32, g.shape, 1)
    gp = g + jnp.where(col == one_col, 1.0, 0.0)
    g_ref[...] = gp
    s_ref[...] = jnp.sum(gp * asv_ref[...], axis=1, keepdims=True)
    d_ref[...] = jnp.sum(gp * adv_ref[...], axis=1, keepdims=True)


def _tc_mid(acc_a, acc_b, b1p, w2p, as2p, ad2p, *, den_col, one_col, br=1000):
    n, wp1 = acc_a.shape
    wp2 = w2p.shape[1]
    grid = (n // br,)
    return pl.pallas_call(
        functools.partial(_mid_body, den_col=den_col, one_col=one_col),
        grid=grid,
        in_specs=[
            pl.BlockSpec((br, wp1), lambda i: (i, 0)),
            pl.BlockSpec((br, wp1), lambda i: (i, 0)),
            pl.BlockSpec((1, wp1), lambda i: (0, 0)),
            pl.BlockSpec((wp1, wp2), lambda i: (0, 0)),
            pl.BlockSpec((1, wp2), lambda i: (0, 0)),
            pl.BlockSpec((1, wp2), lambda i: (0, 0)),
        ],
        out_specs=[
            pl.BlockSpec((br, wp2), lambda i: (i, 0)),
            pl.BlockSpec((br, 1), lambda i: (i, 0)),
            pl.BlockSpec((br, 1), lambda i: (i, 0)),
        ],
        out_shape=[
            jax.ShapeDtypeStruct((n, wp2), jnp.float32),
            jax.ShapeDtypeStruct((n, 1), jnp.float32),
            jax.ShapeDtypeStruct((n, 1), jnp.float32),
        ],
    )(acc_a, acc_b, b1p, w2p, as2p, ad2p)


def _fin_body(a_ref, b_ref, bias_ref, o_ref, *, den_col, ncls):
    s = a_ref[...] + b_ref[...]
    den = s[:, den_col:den_col + 1]
    o_ref[...] = s[:, :ncls] / (den + 1e-16) + bias_ref[...]


def _tc_fin(acc_a, acc_b, b2, *, den_col, ncls, br=1000):
    n, wp = acc_a.shape
    grid = (n // br,)
    return pl.pallas_call(
        functools.partial(_fin_body, den_col=den_col, ncls=ncls),
        grid=grid,
        in_specs=[
            pl.BlockSpec((br, wp), lambda i: (i, 0)),
            pl.BlockSpec((br, wp), lambda i: (i, 0)),
            pl.BlockSpec((1, ncls), lambda i: (0, 0)),
        ],
        out_specs=pl.BlockSpec((br, ncls), lambda i: (i, 0)),
        out_shape=jax.ShapeDtypeStruct((n, ncls), jnp.float32),
    )(acc_a, acc_b, b2)


def _sc_gat(hp, ei3, eal2, asrc, adst, *, wp):
    n = hp.shape[0]
    nch_all, ch = ei3.shape[1:]
    e = nch_all * ch
    ew = e // (_NC * _NS)
    nch = ew // ch
    rb = (n // _NS) // 8 * 8
    zr = rb // 3
    rem = n - _NS * rb
    assert ew % ch == 0 and rb % zr == 0 and ch % _L == 0
    assert rem % 8 == 0 and rem < zr and nch % 2 == 1

    mesh = plsc.VectorSubcoreMesh(core_axis_name="c", subcore_axis_name="s",
                                  num_cores=_NC, num_subcores=_NS)

    @functools.partial(
        pl.kernel,
        out_type=jax.ShapeDtypeStruct((_NC, n, wp), jnp.float32),
        mesh=mesh,
        compiler_params=pltpu.CompilerParams(needs_layout_passes=False,
                                             use_tc_tiling_on_sc=False),
        scratch_types=[
            pltpu.VMEM((n,), jnp.float32),
            pltpu.VMEM((n,), jnp.float32),
            pltpu.VMEM((zr, wp), jnp.float32),
            pltpu.VMEM((nch, ch), jnp.int32),
            pltpu.VMEM((nch, ch), jnp.int32),
            pltpu.VMEM((nch, ch), jnp.float32),
            pltpu.VMEM((nch, ch), jnp.float32),
            pltpu.VMEM((ch, wp), jnp.float32),
            pltpu.VMEM((ch, wp), jnp.float32),
            pltpu.VMEM((ch, wp), jnp.float32),
            pltpu.VMEM_SHARED((n, wp), jnp.float32),
            pltpu.SemaphoreType.DMA,
            pltpu.SemaphoreType.DMA,
            pltpu.SemaphoreType.DMA,
            pltpu.SemaphoreType.DMA,
            pltpu.SemaphoreType.DMA,
            pltpu.SemaphoreType.DMA,
        ],
    )
    def k(hp_h, ei_h, eal_h, asrc_h, adst_h, out_h,
          asrc_v, adst_v, zbuf, srcv, dstv, ealv, exv, rows_0, rows_1,
          rows_2, acc_sh, g_0, g_1, g_2, s_0, s_1, s_2):
        c = lax.axis_index("c")
        s = lax.axis_index("s")

        z16 = jnp.zeros((_L,), jnp.float32)

        def zrow(r, _):
            for q in range(wp // _L):
                zbuf[r, pl.ds(q * _L, _L)] = z16
            return _
        lax.fori_loop(0, zr, zrow, None)

        row0 = s * rb

        def zacc(j, _):
            pltpu.sync_copy(zbuf, acc_sh.at[pl.ds(row0 + j * zr, zr), :])
            return _
        lax.fori_loop(0, rb // zr, zacc, None)

        @pl.when(s == _NS - 1)
        def _():
            pltpu.sync_copy(zbuf.at[pl.ds(0, rem), :],
                            acc_sh.at[pl.ds(_NS * rb, rem), :])

        ch0 = c * (nch_all // _NC) + s * nch
        pltpu.sync_copy(ei_h.at[0, pl.ds(ch0, nch), :], srcv)
        pltpu.sync_copy(ei_h.at[1, pl.ds(ch0, nch), :], dstv)
        pltpu.sync_copy(eal_h.at[pl.ds(ch0, nch), :], ealv)
        pltpu.sync_copy(asrc_h, asrc_v)
        pltpu.sync_copy(adst_h, adst_v)

        def exrow(ci, _):
            for v in range(ch // _L):
                sl = pl.ds(v * _L, _L)
                a = (plsc.load_gather(asrc_v, [srcv[ci, sl]])
                     + plsc.load_gather(adst_v, [dstv[ci, sl]])
                     + ealv[ci, sl])
                a = jnp.where(a >= 0.0, a, 0.2 * a)
                exv[ci, sl] = jnp.exp(a)
            return _
        lax.fori_loop(0, nch, exrow, None)

        plsc.subcore_barrier()

        def scale(ci, rows):
            civ = jnp.full((_L,), ci, jnp.int32)

            def sc8(i, _):
                base = i * 8
                cbs = [plsc.load_gather(
                    exv, [civ, jnp.full((_L,), base + u, jnp.int32)])
                    for u in range(8)]
                for u in range(8):
                    for q in range(wp // _L):
                        sl = pl.ds(q * _L, _L)
                        rows[base + u, sl] = rows[base + u, sl] * cbs[u]
                return _
            lax.fori_loop(0, ch // 8, sc8, None)

        rbufs = (rows_0, rows_1, rows_2)
        gsems = (g_0, g_1, g_2)
        ssems = (s_0, s_1, s_2)

        def gath(ci, i):
            pltpu.async_copy(hp_h.at[srcv.at[ci]], rbufs[i], gsems[i])

        def gwait(i):
            pltpu.make_async_copy(hp_h.at[srcv.at[0]], rbufs[i],
                                  gsems[i]).wait()

        def scat(ci, i):
            pltpu.async_copy(rbufs[i], acc_sh.at[dstv.at[ci]], ssems[i],
                             add=True)

        def swait(i):
            pltpu.make_async_copy(rbufs[i], acc_sh.at[dstv.at[0]],
                                  ssems[i]).wait()

        def step(c, i, first):
            if not first:
                swait((i + 1) % 3)
            gath(c + 1, (i + 1) % 3)
            gwait(i)
            scale(c, rbufs[i])
            scat(c, i)

        gath(0, 0)
        step(0, 0, True)
        step(1, 1, True)

        def trip(m, _):
            c = 3 * m + 2
            step(c, 2, False)
            step(c + 1, 0, False)
            swait(2)
            gath(jnp.minimum(3 * m + 5, nch - 1), 2)
            gwait(1)
            scale(c + 2, rbufs[1])
            scat(c + 2, 1)
            return _
        lax.fori_loop(0, (nch - 2) // 3, trip, None)
        gwait(2)
        swait(0)
        swait(1)

        plsc.subcore_barrier()

        def outc(j, _):
            sl = pl.ds(row0 + j * zr, zr)
            pltpu.sync_copy(acc_sh.at[sl, :], out_h.at[c, sl, :])
            return _
        lax.fori_loop(0, rb // zr, outc, None)

        @pl.when(s == _NS - 1)
        def _():
            sl = pl.ds(_NS * rb, rem)
            pltpu.sync_copy(acc_sh.at[sl, :], out_h.at[c, sl, :])

    return k(hp, ei3, eal2, asrc, adst)


def kernel(x, edge_index, edge_attr, W1, as1, ad1, We1, ae1, b1,
           W2, as2, ad2, We2, ae2, b2):
    n, df = x.shape
    e = edge_index.shape[1]
    h1 = W1.shape[1]
    ncls = W2.shape[1]
    de = edge_attr.shape[1]
    wp1 = 64
    wp2 = 32

    f32 = jnp.float32
    ch = 80

    w1p = jnp.zeros((df, wp1), f32).at[:, :h1].set(W1)
    as1p = jnp.zeros((1, wp1), f32).at[0, :h1].set(as1)
    ad1p = jnp.zeros((1, wp1), f32).at[0, :h1].set(ad1)
    b1p = jnp.zeros((1, wp1), f32).at[0, :h1].set(b1)
    w2p = jnp.zeros((wp1, wp2), f32).at[:h1, :ncls].set(W2)
    as2p = jnp.zeros((1, wp2), f32).at[0, :ncls].set(as2)
    ad2p = jnp.zeros((1, wp2), f32).at[0, :ncls].set(ad2)
    b2r = b2.reshape(1, ncls)
    wae1 = We1 @ ae1
    wae2 = We2 @ ae2
    eyec = jnp.eye(ch, dtype=f32)
    wblk1 = jnp.kron(eyec, wae1.reshape(de, 1))
    wblk2 = jnp.kron(eyec, wae2.reshape(de, 1))
    ea3 = edge_attr.reshape(e // ch, ch * de)

    h1p, asrc1, adst1, eal1c, eal2c, ei3 = _tc_front(
        x, w1p, as1p, ad1p, ea3, wblk1, wblk2, edge_index,
        wp=wp1, one_col=h1, ch=ch)

    acc1 = _sc_gat(h1p, ei3, eal1c, asrc1.reshape(n), adst1.reshape(n),
                   wp=wp1)

    g2p, asrc2, adst2 = _tc_mid(acc1[0], acc1[1], b1p, w2p, as2p, ad2p,
                                den_col=h1, one_col=ncls)

    acc2 = _sc_gat(g2p, ei3, eal2c, asrc2.reshape(n), adst2.reshape(n),
                   wp=wp2)

    return _tc_fin(acc2[0], acc2[1], b2r, den_col=ncls, ncls=ncls)


# speedup vs baseline: 1.0158x; 1.0158x over previous
import functools

import jax
import jax.numpy as jnp
from jax import lax
from jax.experimental import pallas as pl
from jax.experimental.pallas import tpu as pltpu
from jax.experimental.pallas import tpu_sc as plsc

_NC = 2
_NS = 16
_L = 16


def _front_body(x_ref, w_ref, asv_ref, adv_ref, ea_ref, w1b_ref, w2b_ref,
                h_ref, s_ref, d_ref, e1_ref, e2_ref, *, one_col):
    h = jnp.dot(x_ref[...], w_ref[...], preferred_element_type=jnp.float32)
    col = lax.broadcasted_iota(jnp.int32, h.shape, 1)
    hp = h + jnp.where(col == one_col, 1.0, 0.0)
    h_ref[...] = hp
    s_ref[...] = jnp.sum(hp * asv_ref[...], axis=1, keepdims=True)
    d_ref[...] = jnp.sum(hp * adv_ref[...], axis=1, keepdims=True)
    ea = ea_ref[...]
    e1_ref[...] = jnp.dot(ea, w1b_ref[...], preferred_element_type=jnp.float32)
    e2_ref[...] = jnp.dot(ea, w2b_ref[...], preferred_element_type=jnp.float32)


def _tc_front(x, w1p, as1p, ad1p, ea3, wblk1, wblk2, *, wp, one_col,
              br=1000, bre=400):
    n, df = x.shape
    r, c = ea3.shape
    k = wblk1.shape[1]
    grid = (n // br,)
    assert r // bre == n // br
    return pl.pallas_call(
        functools.partial(_front_body, one_col=one_col),
        grid=grid,
        in_specs=[
            pl.BlockSpec((br, df), lambda i: (i, 0)),
            pl.BlockSpec((df, wp), lambda i: (0, 0)),
            pl.BlockSpec((1, wp), lambda i: (0, 0)),
            pl.BlockSpec((1, wp), lambda i: (0, 0)),
            pl.BlockSpec((bre, c), lambda i: (i, 0)),
            pl.BlockSpec((c, k), lambda i: (0, 0)),
            pl.BlockSpec((c, k), lambda i: (0, 0)),
        ],
        out_specs=[
            pl.BlockSpec((br, wp), lambda i: (i, 0)),
            pl.BlockSpec((br, 1), lambda i: (i, 0)),
            pl.BlockSpec((br, 1), lambda i: (i, 0)),
            pl.BlockSpec((bre, k), lambda i: (i, 0)),
            pl.BlockSpec((bre, k), lambda i: (i, 0)),
        ],
        out_shape=[
            jax.ShapeDtypeStruct((n, wp), jnp.float32),
            jax.ShapeDtypeStruct((n, 1), jnp.float32),
            jax.ShapeDtypeStruct((n, 1), jnp.float32),
            jax.ShapeDtypeStruct((r, k), jnp.float32),
            jax.ShapeDtypeStruct((r, k), jnp.float32),
        ],
    )(x, w1p, as1p, ad1p, ea3, wblk1, wblk2)


def _mid_body(a_ref, b_ref, bias_ref, w2_ref, asv_ref, adv_ref,
              g_ref, s_ref, d_ref, *, den_col, one_col):
    s = a_ref[...] + b_ref[...]
    den = s[:, den_col:den_col + 1]
    o = s / (den + 1e-16)
    h2 = jnp.maximum(o + bias_ref[...], 0.0)
    g = jnp.dot(h2, w2_ref[...], preferred_element_type=jnp.float32)
    col = lax.broadcasted_iota(jnp.int32, g.shape, 1)
    gp = g + jnp.where(col == one_col, 1.0, 0.0)
    g_ref[...] = gp
    s_ref[...] = jnp.sum(gp * asv_ref[...], axis=1, keepdims=True)
    d_ref[...] = jnp.sum(gp * adv_ref[...], axis=1, keepdims=True)


def _tc_mid(acc_a, acc_b, b1p, w2p, as2p, ad2p, *, den_col, one_col, br=1000):
    n, wp1 = acc_a.shape
    wp2 = w2p.shape[1]
    grid = (n // br,)
    return pl.pallas_call(
        functools.partial(_mid_body, den_col=den_col, one_col=one_col),
        grid=grid,
        in_specs=[
            pl.BlockSpec((br, wp1), lambda i: (i, 0)),
            pl.BlockSpec((br, wp1), lambda i: (i, 0)),
            pl.BlockSpec((1, wp1), lambda i: (0, 0)),
            pl.BlockSpec((wp1, wp2), lambda i: (0, 0)),
            pl.BlockSpec((1, wp2), lambda i: (0, 0)),
            pl.BlockSpec((1, wp2), lambda i: (0, 0)),
        ],
        out_specs=[
            pl.BlockSpec((br, wp2), lambda i: (i, 0)),
            pl.BlockSpec((br, 1), lambda i: (i, 0)),
            pl.BlockSpec((br, 1), lambda i: (i, 0)),
        ],
        out_shape=[
            jax.ShapeDtypeStruct((n, wp2), jnp.float32),
            jax.ShapeDtypeStruct((n, 1), jnp.float32),
            jax.ShapeDtypeStruct((n, 1), jnp.float32),
        ],
    )(acc_a, acc_b, b1p, w2p, as2p, ad2p)


def _fin_body(a_ref, b_ref, bias_ref, o_ref, *, den_col, ncls):
    s = a_ref[...] + b_ref[...]
    den = s[:, den_col:den_col + 1]
    o_ref[...] = s[:, :ncls] / (den + 1e-16) + bias_ref[...]


def _tc_fin(acc_a, acc_b, b2, *, den_col, ncls, br=1000):
    n, wp = acc_a.shape
    grid = (n // br,)
    return pl.pallas_call(
        functools.partial(_fin_body, den_col=den_col, ncls=ncls),
        grid=grid,
        in_specs=[
            pl.BlockSpec((br, wp), lambda i: (i, 0)),
            pl.BlockSpec((br, wp), lambda i: (i, 0)),
            pl.BlockSpec((1, ncls), lambda i: (0, 0)),
        ],
        out_specs=pl.BlockSpec((br, ncls), lambda i: (i, 0)),
        out_shape=jax.ShapeDtypeStruct((n, ncls), jnp.float32),
    )(acc_a, acc_b, b2)


def _sc_gat(hp, ei3, eal2, asrc, adst, *, wp):
    n = hp.shape[0]
    nch_all, ch = ei3.shape[1:]
    e = nch_all * ch
    ew = e // (_NC * _NS)
    nch = ew // ch
    rb = (n // _NS) // 8 * 8
    zr = rb // 3
    rem = n - _NS * rb
    assert ew % ch == 0 and rb % zr == 0 and ch % _L == 0
    assert rem % 8 == 0 and rem < zr and nch % 2 == 1

    mesh = plsc.VectorSubcoreMesh(core_axis_name="c", subcore_axis_name="s",
                                  num_cores=_NC, num_subcores=_NS)

    @functools.partial(
        pl.kernel,
        out_type=jax.ShapeDtypeStruct((_NC, n, wp), jnp.float32),
        mesh=mesh,
        compiler_params=pltpu.CompilerParams(needs_layout_passes=False,
                                             use_tc_tiling_on_sc=False),
        scratch_types=[
            pltpu.VMEM((n,), jnp.float32),
            pltpu.VMEM((n,), jnp.float32),
            pltpu.VMEM((zr, wp), jnp.float32),
            pltpu.VMEM((nch, ch), jnp.int32),
            pltpu.VMEM((nch, ch), jnp.int32),
            pltpu.VMEM((nch, ch), jnp.float32),
            pltpu.VMEM((nch, ch), jnp.float32),
            pltpu.VMEM((ch, wp), jnp.float32),
            pltpu.VMEM((ch, wp), jnp.float32),
            pltpu.VMEM((ch, wp), jnp.float32),
            pltpu.VMEM_SHARED((n, wp), jnp.float32),
            pltpu.SemaphoreType.DMA,
            pltpu.SemaphoreType.DMA,
            pltpu.SemaphoreType.DMA,
            pltpu.SemaphoreType.DMA,
            pltpu.SemaphoreType.DMA,
            pltpu.SemaphoreType.DMA,
        ],
    )
    def k(hp_h, ei_h, eal_h, asrc_h, adst_h, out_h,
          asrc_v, adst_v, zbuf, srcv, dstv, ealv, exv, rows_0, rows_1,
          rows_2, acc_sh, g_0, g_1, g_2, s_0, s_1, s_2):
        c = lax.axis_index("c")
        s = lax.axis_index("s")

        z16 = jnp.zeros((_L,), jnp.float32)

        def zrow(r, _):
            for q in range(wp // _L):
                zbuf[r, pl.ds(q * _L, _L)] = z16
            return _
        lax.fori_loop(0, zr, zrow, None)

        row0 = s * rb

        def zacc(j, _):
            pltpu.sync_copy(zbuf, acc_sh.at[pl.ds(row0 + j * zr, zr), :])
            return _
        lax.fori_loop(0, rb // zr, zacc, None)

        @pl.when(s == _NS - 1)
        def _():
            pltpu.sync_copy(zbuf.at[pl.ds(0, rem), :],
                            acc_sh.at[pl.ds(_NS * rb, rem), :])

        ch0 = c * (nch_all // _NC) + s * nch
        pltpu.sync_copy(ei_h.at[0, pl.ds(ch0, nch), :], srcv)
        pltpu.sync_copy(ei_h.at[1, pl.ds(ch0, nch), :], dstv)
        pltpu.sync_copy(eal_h.at[pl.ds(ch0, nch), :], ealv)
        pltpu.sync_copy(asrc_h, asrc_v)
        pltpu.sync_copy(adst_h, adst_v)

        def exrow(ci, _):
            for v in range(ch // _L):
                sl = pl.ds(v * _L, _L)
                a = (plsc.load_gather(asrc_v, [srcv[ci, sl]])
                     + plsc.load_gather(adst_v, [dstv[ci, sl]])
                     + ealv[ci, sl])
                a = jnp.where(a >= 0.0, a, 0.2 * a)
                exv[ci, sl] = jnp.exp(a)
            return _
        lax.fori_loop(0, nch, exrow, None)

        plsc.subcore_barrier()

        def scale(ci, rows):
            civ = jnp.full((_L,), ci, jnp.int32)

            def sc8(i, _):
                base = i * 8
                cbs = [plsc.load_gather(
                    exv, [civ, jnp.full((_L,), base + u, jnp.int32)])
                    for u in range(8)]
                for u in range(8):
                    for q in range(wp // _L):
                        sl = pl.ds(q * _L, _L)
                        rows[base + u, sl] = rows[base + u, sl] * cbs[u]
                return _
            lax.fori_loop(0, ch // 8, sc8, None)

        rbufs = (rows_0, rows_1, rows_2)
        gsems = (g_0, g_1, g_2)
        ssems = (s_0, s_1, s_2)

        def gath(ci, i):
            pltpu.async_copy(hp_h.at[srcv.at[ci]], rbufs[i], gsems[i])

        def gwait(i):
            pltpu.make_async_copy(hp_h.at[srcv.at[0]], rbufs[i],
                                  gsems[i]).wait()

        def scat(ci, i):
            pltpu.async_copy(rbufs[i], acc_sh.at[dstv.at[ci]], ssems[i],
                             add=True)

        def swait(i):
            pltpu.make_async_copy(rbufs[i], acc_sh.at[dstv.at[0]],
                                  ssems[i]).wait()

        def step(c, i, first):
            if not first:
                swait((i + 1) % 3)
            gath(c + 1, (i + 1) % 3)
            gwait(i)
            scale(c, rbufs[i])
            scat(c, i)

        gath(0, 0)
        step(0, 0, True)
        step(1, 1, True)

        def trip(m, _):
            c = 3 * m + 2
            step(c, 2, False)
            step(c + 1, 0, False)
            swait(2)
            gath(jnp.minimum(3 * m + 5, nch - 1), 2)
            gwait(1)
            scale(c + 2, rbufs[1])
            scat(c + 2, 1)
            return _
        lax.fori_loop(0, (nch - 2) // 3, trip, None)
        gwait(2)
        swait(0)
        swait(1)

        plsc.subcore_barrier()

        def outc(j, _):
            sl = pl.ds(row0 + j * zr, zr)
            pltpu.sync_copy(acc_sh.at[sl, :], out_h.at[c, sl, :])
            return _
        lax.fori_loop(0, rb // zr, outc, None)

        @pl.when(s == _NS - 1)
        def _():
            sl = pl.ds(_NS * rb, rem)
            pltpu.sync_copy(acc_sh.at[sl, :], out_h.at[c, sl, :])

    return k(hp, ei3, eal2, asrc, adst)


def kernel(x, edge_index, edge_attr, W1, as1, ad1, We1, ae1, b1,
           W2, as2, ad2, We2, ae2, b2):
    n, df = x.shape
    e = edge_index.shape[1]
    h1 = W1.shape[1]
    ncls = W2.shape[1]
    de = edge_attr.shape[1]
    wp1 = 64
    wp2 = 32

    f32 = jnp.float32
    ch = 80
    ei3 = edge_index.reshape(2, e // ch, ch)

    w1p = jnp.zeros((df, wp1), f32).at[:, :h1].set(W1)
    as1p = jnp.zeros((1, wp1), f32).at[0, :h1].set(as1)
    ad1p = jnp.zeros((1, wp1), f32).at[0, :h1].set(ad1)
    b1p = jnp.zeros((1, wp1), f32).at[0, :h1].set(b1)
    w2p = jnp.zeros((wp1, wp2), f32).at[:h1, :ncls].set(W2)
    as2p = jnp.zeros((1, wp2), f32).at[0, :ncls].set(as2)
    ad2p = jnp.zeros((1, wp2), f32).at[0, :ncls].set(ad2)
    b2r = b2.reshape(1, ncls)
    wae1 = We1 @ ae1
    wae2 = We2 @ ae2
    eyec = jnp.eye(ch, dtype=f32)
    wblk1 = jnp.kron(eyec, wae1.reshape(de, 1))
    wblk2 = jnp.kron(eyec, wae2.reshape(de, 1))
    ea3 = edge_attr.reshape(e // ch, ch * de)

    h1p, asrc1, adst1, eal1c, eal2c = _tc_front(
        x, w1p, as1p, ad1p, ea3, wblk1, wblk2, wp=wp1, one_col=h1)

    acc1 = _sc_gat(h1p, ei3, eal1c, asrc1.reshape(n), adst1.reshape(n),
                   wp=wp1)

    g2p, asrc2, adst2 = _tc_mid(acc1[0], acc1[1], b1p, w2p, as2p, ad2p,
                                den_col=h1, one_col=ncls)

    acc2 = _sc_gat(g2p, ei3, eal2c, asrc2.reshape(n), adst2.reshape(n),
                   wp=wp2)

    return _tc_fin(acc2[0], acc2[1], b2r, den_col=ncls, ncls=ncls)


# speedup vs baseline: 1.0337x; 1.0177x over previous
import functools

import jax
import jax.numpy as jnp
from jax import lax
from jax.experimental import pallas as pl
from jax.experimental.pallas import tpu as pltpu
from jax.experimental.pallas import tpu_sc as plsc

_NC = 2
_NS = 16
_L = 16


def _front_body(x_ref, w_ref, asv_ref, adv_ref, ea_ref, w1b_ref, w2b_ref,
                h_ref, s_ref, d_ref, e1_ref, e2_ref, *, one_col):
    h = jnp.dot(x_ref[...], w_ref[...], preferred_element_type=jnp.float32)
    col = lax.broadcasted_iota(jnp.int32, h.shape, 1)
    hp = h + jnp.where(col == one_col, 1.0, 0.0)
    h_ref[...] = hp
    s_ref[...] = jnp.sum(hp * asv_ref[...], axis=1, keepdims=True)
    d_ref[...] = jnp.sum(hp * adv_ref[...], axis=1, keepdims=True)
    ea = ea_ref[...]
    e1_ref[...] = jnp.dot(ea, w1b_ref[...], preferred_element_type=jnp.float32)
    e2_ref[...] = jnp.dot(ea, w2b_ref[...], preferred_element_type=jnp.float32)


def _tc_front(x, w1p, as1p, ad1p, ea3, wblk1, wblk2, *, wp, one_col,
              br=1000, bre=400):
    n, df = x.shape
    r, c = ea3.shape
    k = wblk1.shape[1]
    grid = (n // br,)
    assert r // bre == n // br
    return pl.pallas_call(
        functools.partial(_front_body, one_col=one_col),
        grid=grid,
        in_specs=[
            pl.BlockSpec((br, df), lambda i: (i, 0)),
            pl.BlockSpec((df, wp), lambda i: (0, 0)),
            pl.BlockSpec((1, wp), lambda i: (0, 0)),
            pl.BlockSpec((1, wp), lambda i: (0, 0)),
            pl.BlockSpec((bre, c), lambda i: (i, 0)),
            pl.BlockSpec((c, k), lambda i: (0, 0)),
            pl.BlockSpec((c, k), lambda i: (0, 0)),
        ],
        out_specs=[
            pl.BlockSpec((br, wp), lambda i: (i, 0)),
            pl.BlockSpec((br, 1), lambda i: (i, 0)),
            pl.BlockSpec((br, 1), lambda i: (i, 0)),
            pl.BlockSpec((bre, k), lambda i: (i, 0)),
            pl.BlockSpec((bre, k), lambda i: (i, 0)),
        ],
        out_shape=[
            jax.ShapeDtypeStruct((n, wp), jnp.float32),
            jax.ShapeDtypeStruct((n, 1), jnp.float32),
            jax.ShapeDtypeStruct((n, 1), jnp.float32),
            jax.ShapeDtypeStruct((r, k), jnp.float32),
            jax.ShapeDtypeStruct((r, k), jnp.float32),
        ],
    )(x, w1p, as1p, ad1p, ea3, wblk1, wblk2)


def _mid_body(a_ref, b_ref, bias_ref, w2_ref, asv_ref, adv_ref,
              g_ref, s_ref, d_ref, *, den_col, one_col):
    s = a_ref[...] + b_ref[...]
    den = s[:, den_col:den_col + 1]
    o = s / (den + 1e-16)
    h2 = jnp.maximum(o + bias_ref[...], 0.0)
    g = jnp.dot(h2, w2_ref[...], preferred_element_type=jnp.float32)
    col = lax.broadcasted_iota(jnp.int32, g.shape, 1)
    gp = g + jnp.where(col == one_col, 1.0, 0.0)
    g_ref[...] = gp
    s_ref[...] = jnp.sum(gp * asv_ref[...], axis=1, keepdims=True)
    d_ref[...] = jnp.sum(gp * adv_ref[...], axis=1, keepdims=True)


def _tc_mid(acc_a, acc_b, b1p, w2p, as2p, ad2p, *, den_col, one_col, br=1000):
    n, wp1 = acc_a.shape
    wp2 = w2p.shape[1]
    grid = (n // br,)
    return pl.pallas_call(
        functools.partial(_mid_body, den_col=den_col, one_col=one_col),
        grid=grid,
        in_specs=[
            pl.BlockSpec((br, wp1), lambda i: (i, 0)),
            pl.BlockSpec((br, wp1), lambda i: (i, 0)),
            pl.BlockSpec((1, wp1), lambda i: (0, 0)),
            pl.BlockSpec((wp1, wp2), lambda i: (0, 0)),
            pl.BlockSpec((1, wp2), lambda i: (0, 0)),
            pl.BlockSpec((1, wp2), lambda i: (0, 0)),
        ],
        out_specs=[
            pl.BlockSpec((br, wp2), lambda i: (i, 0)),
            pl.BlockSpec((br, 1), lambda i: (i, 0)),
            pl.BlockSpec((br, 1), lambda i: (i, 0)),
        ],
        out_shape=[
            jax.ShapeDtypeStruct((n, wp2), jnp.float32),
            jax.ShapeDtypeStruct((n, 1), jnp.float32),
            jax.ShapeDtypeStruct((n, 1), jnp.float32),
        ],
    )(acc_a, acc_b, b1p, w2p, as2p, ad2p)


def _fin_body(a_ref, b_ref, bias_ref, o_ref, *, den_col, ncls):
    s = a_ref[...] + b_ref[...]
    den = s[:, den_col:den_col + 1]
    o_ref[...] = s[:, :ncls] / (den + 1e-16) + bias_ref[...]


def _tc_fin(acc_a, acc_b, b2, *, den_col, ncls, br=1000):
    n, wp = acc_a.shape
    grid = (n // br,)
    return pl.pallas_call(
        functools.partial(_fin_body, den_col=den_col, ncls=ncls),
        grid=grid,
        in_specs=[
            pl.BlockSpec((br, wp), lambda i: (i, 0)),
            pl.BlockSpec((br, wp), lambda i: (i, 0)),
            pl.BlockSpec((1, ncls), lambda i: (0, 0)),
        ],
        out_specs=pl.BlockSpec((br, ncls), lambda i: (i, 0)),
        out_shape=jax.ShapeDtypeStruct((n, ncls), jnp.float32),
    )(acc_a, acc_b, b2)


def _sc_gat(hp, ei3, eal2, asrc, adst, *, wp):
    n = hp.shape[0]
    nch_all, ch = ei3.shape[1:]
    e = nch_all * ch
    ew = e // (_NC * _NS)
    nch = ew // ch
    rb = (n // _NS) // 8 * 8
    zr = rb // 3
    rem = n - _NS * rb
    assert ew % ch == 0 and rb % zr == 0 and ch % _L == 0
    assert rem % 8 == 0 and rem < zr and nch % 2 == 1

    mesh = plsc.VectorSubcoreMesh(core_axis_name="c", subcore_axis_name="s",
                                  num_cores=_NC, num_subcores=_NS)

    @functools.partial(
        pl.kernel,
        out_type=jax.ShapeDtypeStruct((_NC, n, wp), jnp.float32),
        mesh=mesh,
        compiler_params=pltpu.CompilerParams(needs_layout_passes=False,
                                             use_tc_tiling_on_sc=False),
        scratch_types=[
            pltpu.VMEM((n,), jnp.float32),
            pltpu.VMEM((n,), jnp.float32),
            pltpu.VMEM((zr, wp), jnp.float32),
            pltpu.VMEM((nch, ch), jnp.int32),
            pltpu.VMEM((nch, ch), jnp.int32),
            pltpu.VMEM((nch, ch), jnp.float32),
            pltpu.VMEM((nch, ch), jnp.float32),
            pltpu.VMEM((ch, wp), jnp.float32),
            pltpu.VMEM((ch, wp), jnp.float32),
            pltpu.VMEM((ch, wp), jnp.float32),
            pltpu.VMEM_SHARED((n, wp), jnp.float32),
            pltpu.SemaphoreType.DMA,
            pltpu.SemaphoreType.DMA,
            pltpu.SemaphoreType.DMA,
            pltpu.SemaphoreType.DMA,
            pltpu.SemaphoreType.DMA,
            pltpu.SemaphoreType.DMA,
        ],
    )
    def k(hp_h, ei_h, eal_h, asrc_h, adst_h, out_h,
          asrc_v, adst_v, zbuf, srcv, dstv, ealv, exv, rows_0, rows_1,
          rows_2, acc_sh, g_0, g_1, g_2, s_0, s_1, s_2):
        c = lax.axis_index("c")
        s = lax.axis_index("s")

        ch0 = c * (nch_all // _NC) + s * nch
        st = (
            pltpu.async_copy(ei_h.at[0, pl.ds(ch0, nch), :], srcv, g_0),
            pltpu.async_copy(ei_h.at[1, pl.ds(ch0, nch), :], dstv, g_1),
            pltpu.async_copy(eal_h.at[pl.ds(ch0, nch), :], ealv, g_2),
            pltpu.async_copy(asrc_h, asrc_v, s_0),
            pltpu.async_copy(adst_h, adst_v, s_1),
        )

        z16 = jnp.zeros((_L,), jnp.float32)

        def zrow(r, _):
            for q in range(wp // _L):
                zbuf[r, pl.ds(q * _L, _L)] = z16
            return _
        lax.fori_loop(0, zr, zrow, None)

        row0 = s * rb

        def zacc(j, _):
            pltpu.sync_copy(zbuf, acc_sh.at[pl.ds(row0 + j * zr, zr), :])
            return _
        lax.fori_loop(0, rb // zr, zacc, None)

        @pl.when(s == _NS - 1)
        def _():
            pltpu.sync_copy(zbuf.at[pl.ds(0, rem), :],
                            acc_sh.at[pl.ds(_NS * rb, rem), :])

        for cp in st:
            cp.wait()

        def exrow(ci, _):
            for v in range(ch // _L):
                sl = pl.ds(v * _L, _L)
                a = (plsc.load_gather(asrc_v, [srcv[ci, sl]])
                     + plsc.load_gather(adst_v, [dstv[ci, sl]])
                     + ealv[ci, sl])
                a = jnp.where(a >= 0.0, a, 0.2 * a)
                exv[ci, sl] = jnp.exp(a)
            return _
        lax.fori_loop(0, nch, exrow, None)

        plsc.subcore_barrier()

        def scale(ci, rows):
            civ = jnp.full((_L,), ci, jnp.int32)

            def sc8(i, _):
                base = i * 8
                cbs = [plsc.load_gather(
                    exv, [civ, jnp.full((_L,), base + u, jnp.int32)])
                    for u in range(8)]
                for u in range(8):
                    for q in range(wp // _L):
                        sl = pl.ds(q * _L, _L)
                        rows[base + u, sl] = rows[base + u, sl] * cbs[u]
                return _
            lax.fori_loop(0, ch // 8, sc8, None)

        rbufs = (rows_0, rows_1, rows_2)
        gsems = (g_0, g_1, g_2)
        ssems = (s_0, s_1, s_2)

        def gath(ci, i):
            pltpu.async_copy(hp_h.at[srcv.at[ci]], rbufs[i], gsems[i])

        def gwait(i):
            pltpu.make_async_copy(hp_h.at[srcv.at[0]], rbufs[i],
                                  gsems[i]).wait()

        def scat(ci, i):
            pltpu.async_copy(rbufs[i], acc_sh.at[dstv.at[ci]], ssems[i],
                             add=True)

        def swait(i):
            pltpu.make_async_copy(rbufs[i], acc_sh.at[dstv.at[0]],
                                  ssems[i]).wait()

        def step(c, i, first):
            if not first:
                swait((i + 1) % 3)
            gath(c + 1, (i + 1) % 3)
            gwait(i)
            scale(c, rbufs[i])
            scat(c, i)

        gath(0, 0)
        step(0, 0, True)
        step(1, 1, True)

        def trip(m, _):
            c = 3 * m + 2
            step(c, 2, False)
            step(c + 1, 0, False)
            swait(2)
            gath(jnp.minimum(3 * m + 5, nch - 1), 2)
            gwait(1)
            scale(c + 2, rbufs[1])
            scat(c + 2, 1)
            return _
        lax.fori_loop(0, (nch - 2) // 3, trip, None)
        gwait(2)
        swait(0)
        swait(1)

        plsc.subcore_barrier()

        def outc(j, _):
            sl = pl.ds(row0 + j * zr, zr)
            pltpu.sync_copy(acc_sh.at[sl, :], out_h.at[c, sl, :])
            return _
        lax.fori_loop(0, rb // zr, outc, None)

        @pl.when(s == _NS - 1)
        def _():
            sl = pl.ds(_NS * rb, rem)
            pltpu.sync_copy(acc_sh.at[sl, :], out_h.at[c, sl, :])

    return k(hp, ei3, eal2, asrc, adst)


def kernel(x, edge_index, edge_attr, W1, as1, ad1, We1, ae1, b1,
           W2, as2, ad2, We2, ae2, b2):
    n, df = x.shape
    e = edge_index.shape[1]
    h1 = W1.shape[1]
    ncls = W2.shape[1]
    de = edge_attr.shape[1]
    wp1 = 64
    wp2 = 32

    f32 = jnp.float32
    ch = 80
    ei3 = edge_index.reshape(2, e // ch, ch)

    w1p = jnp.zeros((df, wp1), f32).at[:, :h1].set(W1)
    as1p = jnp.zeros((1, wp1), f32).at[0, :h1].set(as1)
    ad1p = jnp.zeros((1, wp1), f32).at[0, :h1].set(ad1)
    b1p = jnp.zeros((1, wp1), f32).at[0, :h1].set(b1)
    w2p = jnp.zeros((wp1, wp2), f32).at[:h1, :ncls].set(W2)
    as2p = jnp.zeros((1, wp2), f32).at[0, :ncls].set(as2)
    ad2p = jnp.zeros((1, wp2), f32).at[0, :ncls].set(ad2)
    b2r = b2.reshape(1, ncls)
    wae1 = We1 @ ae1
    wae2 = We2 @ ae2
    eyec = jnp.eye(ch, dtype=f32)
    wblk1 = jnp.kron(eyec, wae1.reshape(de, 1))
    wblk2 = jnp.kron(eyec, wae2.reshape(de, 1))
    ea3 = edge_attr.reshape(e // ch, ch * de)

    h1p, asrc1, adst1, eal1c, eal2c = _tc_front(
        x, w1p, as1p, ad1p, ea3, wblk1, wblk2, wp=wp1, one_col=h1)

    acc1 = _sc_gat(h1p, ei3, eal1c, asrc1.reshape(n), adst1.reshape(n),
                   wp=wp1)

    g2p, asrc2, adst2 = _tc_mid(acc1[0], acc1[1], b1p, w2p, as2p, ad2p,
                                den_col=h1, one_col=ncls)

    acc2 = _sc_gat(g2p, ei3, eal2c, asrc2.reshape(n), adst2.reshape(n),
                   wp=wp2)

    return _tc_fin(acc2[0], acc2[1], b2r, den_col=ncls, ncls=ncls)
